# Initial kernel scaffold; baseline (speedup 1.0000x reference)
import jax
import jax.numpy as jnp
from jax.experimental import pallas as pl


def kernel(x, edge_index, edge_attr, batch, params):
    raise NotImplementedError("write your pallas kernel here")


# speedup vs baseline: 2.9620x; 2.9620x over previous
import functools

import jax
import jax.numpy as jnp
from jax import lax
from jax.experimental import pallas as pl
from jax.experimental.pallas import tpu as pltpu
from jax.experimental.pallas import tpu_sc as plsc

F32 = jnp.float32
NCORE = 2
NSUB = 16
NT = NCORE * NSUB
CK = 80
NB = 2000
EB = 4000
DW = 16


def _mm_bias_body(x_ref, w_ref, b_ref, o_ref):
    o_ref[...] = (
        jnp.dot(x_ref[...], w_ref[...], preferred_element_type=F32)
        + b_ref[...]
    )


def _mm_bias(x, w, b, rows_blk):
    n, kdim = x.shape
    fo = w.shape[1]
    return pl.pallas_call(
        _mm_bias_body,
        grid=(n // rows_blk,),
        in_specs=[
            pl.BlockSpec((rows_blk, kdim), lambda i: (i, 0)),
            pl.BlockSpec((kdim, fo), lambda i: (0, 0)),
            pl.BlockSpec((1, fo), lambda i: (0, 0)),
        ],
        out_specs=pl.BlockSpec((rows_blk, fo), lambda i: (i, 0)),
        out_shape=jax.ShapeDtypeStruct((n, fo), F32),
    )(x, w, b.reshape(1, fo))


def _ab_body(h_ref, wi_ref, wj_ref, a_ref, b_ref):
    h = h_ref[...]
    a_ref[...] = jnp.dot(h, wi_ref[...], preferred_element_type=F32)
    b_ref[...] = jnp.dot(h, wj_ref[...], preferred_element_type=F32)


def _ab(h, wi, wj):
    n, d = h.shape
    sds = jax.ShapeDtypeStruct((n, d), F32)
    return pl.pallas_call(
        _ab_body,
        grid=(n // NB,),
        in_specs=[
            pl.BlockSpec((NB, d), lambda i: (i, 0)),
            pl.BlockSpec((d, d), lambda i: (0, 0)),
            pl.BlockSpec((d, d), lambda i: (0, 0)),
        ],
        out_specs=[
            pl.BlockSpec((NB, d), lambda i: (i, 0)),
            pl.BlockSpec((NB, d), lambda i: (i, 0)),
        ],
        out_shape=[sds, sds],
    )(h, wi, wj)


def _node_body(h_ref, s0_ref, s1_ref, d0_ref, d1_ref, ew2_ref, eb2_ref,
               w1a_ref, w1b_ref, ub1_ref, w2_ref, ub2_ref, o_ref):
    s = s0_ref[...] + s1_ref[...]
    deg = (d0_ref[...] + d1_ref[...])[:, 0:1]
    aggr = (jnp.dot(s, ew2_ref[...], preferred_element_type=F32)
            + deg * eb2_ref[...])
    t = (jnp.dot(h_ref[...], w1a_ref[...], preferred_element_type=F32)
         + jnp.dot(aggr, w1b_ref[...], preferred_element_type=F32)
         + ub1_ref[...])
    t = t * jax.nn.sigmoid(t)
    y = jnp.dot(t, w2_ref[...], preferred_element_type=F32) + ub2_ref[...]
    o_ref[...] = y * jax.nn.sigmoid(y)


def _node_update(h, s0, s1, d0, d1, ew2, eb2, w1a, w1b, ub1, w2, ub2):
    n, d = h.shape
    return pl.pallas_call(
        _node_body,
        grid=(n // NB,),
        in_specs=[
            pl.BlockSpec((NB, d), lambda i: (i, 0)),
            pl.BlockSpec((NB, d), lambda i: (i, 0)),
            pl.BlockSpec((NB, d), lambda i: (i, 0)),
            pl.BlockSpec((NB, DW), lambda i: (i, 0)),
            pl.BlockSpec((NB, DW), lambda i: (i, 0)),
            pl.BlockSpec((d, d), lambda i: (0, 0)),
            pl.BlockSpec((1, d), lambda i: (0, 0)),
            pl.BlockSpec((d, d), lambda i: (0, 0)),
            pl.BlockSpec((d, d), lambda i: (0, 0)),
            pl.BlockSpec((1, d), lambda i: (0, 0)),
            pl.BlockSpec((d, d), lambda i: (0, 0)),
            pl.BlockSpec((1, d), lambda i: (0, 0)),
        ],
        out_specs=pl.BlockSpec((NB, d), lambda i: (i, 0)),
        out_shape=jax.ShapeDtypeStruct((n, d), F32),
    )(h, s0, s1, d0, d1, ew2, eb2.reshape(1, d), w1a, w1b,
      ub1.reshape(1, d), w2, ub2.reshape(1, d))


def _pool_body(batch_ref, h_ref, r1_ref, rb1_ref, r2_ref, rb2_ref, r3_ref,
               rb3_ref, o_ref, acc_ref, cnt_ref):
    i = pl.program_id(0)
    ng = acc_ref.shape[0]

    @pl.when(i == 0)
    def _():
        acc_ref[...] = jnp.zeros_like(acc_ref)
        cnt_ref[...] = jnp.zeros_like(cnt_ref)

    b = batch_ref[...].reshape(1, NB)
    mask = (lax.broadcasted_iota(jnp.int32, (ng, NB), 0) == b).astype(F32)
    acc_ref[...] += lax.dot_general(
        mask, h_ref[...], (((1,), (0,)), ((), ())),
        preferred_element_type=F32)
    cnt_ref[...] += jnp.sum(mask, axis=1, keepdims=True)

    @pl.when(i == pl.num_programs(0) - 1)
    def _():
        pooled = acc_ref[...] / jnp.maximum(cnt_ref[...], 1.0)
        t = jnp.dot(pooled, r1_ref[...], preferred_element_type=F32) + rb1_ref[...]
        t = t * jax.nn.sigmoid(t)
        t = jnp.dot(t, r2_ref[...], preferred_element_type=F32) + rb2_ref[...]
        t = t * jax.nn.sigmoid(t)
        o_ref[...] = jnp.dot(t, r3_ref[...], preferred_element_type=F32) + rb3_ref[...]


def _pool_readout(batch3, h, ng, r1, rb1, r2, rb2, r3, rb3):
    n, d = h.shape
    d2 = r2.shape[1]
    do = r3.shape[1]
    return pl.pallas_call(
        _pool_body,
        grid=(n // NB,),
        in_specs=[
            pl.BlockSpec((1, 1, NB), lambda i: (i, 0, 0)),
            pl.BlockSpec((NB, d), lambda i: (i, 0)),
            pl.BlockSpec((d, d), lambda i: (0, 0)),
            pl.BlockSpec((1, d), lambda i: (0, 0)),
            pl.BlockSpec((d, d2), lambda i: (0, 0)),
            pl.BlockSpec((1, d2), lambda i: (0, 0)),
            pl.BlockSpec((d2, do), lambda i: (0, 0)),
            pl.BlockSpec((1, do), lambda i: (0, 0)),
        ],
        out_specs=pl.BlockSpec((ng, do), lambda i: (0, 0)),
        out_shape=jax.ShapeDtypeStruct((ng, do), F32),
        scratch_shapes=[
            pltpu.VMEM((ng, d), F32),
            pltpu.VMEM((ng, 1), F32),
        ],
    )(batch3, h, r1, rb1.reshape(1, d), r2, rb2.reshape(1, d2), r3,
      rb3.reshape(1, do))


def _sc_mesh():
    return plsc.VectorSubcoreMesh(
        core_axis_name="c", subcore_axis_name="s",
        num_cores=NCORE, num_subcores=NSUB)


def _sc_edge_pass(a_tab, b_tab, e4, dst3, src3):
    nt, nchunk, ck = dst3.shape
    nn, d = a_tab.shape
    rows_per_tile = nn // NT
    zrows = 25

    @functools.partial(
        pl.kernel,
        out_type=jax.ShapeDtypeStruct((NCORE, nn, d), F32),
        mesh=_sc_mesh(),
        compiler_params=pltpu.CompilerParams(use_tc_tiling_on_sc=False),
        scratch_types=[
            pltpu.VMEM((nchunk, ck), jnp.int32),
            pltpu.VMEM((nchunk, ck), jnp.int32),
            pltpu.VMEM((ck, 128), F32),
            pltpu.VMEM((zrows, 128), F32),
            pltpu.VMEM_SHARED((nn, 128), F32),
            pltpu.SemaphoreType.DMA,
        ],
    )
    def k(a_hbm, b_hbm, e_hbm, dst_hbm, src_hbm, out_hbm,
          dst_v, src_v, buf, zbuf, aggr_sh, sem):
        c = lax.axis_index("c")
        s = lax.axis_index("s")
        wid = c * NSUB + s

        def zrow(r, carry):
            for cc in range(8):
                zbuf[r, pl.ds(cc * 16, 16)] = jnp.zeros((16,), F32)
            return carry
        lax.fori_loop(0, zrows, zrow, 0)

        base = s * rows_per_tile

        def zcp(p, carry):
            pltpu.sync_copy(zbuf, aggr_sh.at[pl.ds(base + p * zrows, zrows)])
            return carry
        lax.fori_loop(0, rows_per_tile // zrows, zcp, 0)

        pltpu.sync_copy(dst_hbm.at[wid], dst_v)
        pltpu.sync_copy(src_hbm.at[wid], src_v)
        plsc.subcore_barrier()

        def chunk(j, carry):
            di = dst_v.at[j]
            si = src_v.at[j]
            pltpu.sync_copy(e_hbm.at[wid, j], buf)
            pltpu.async_copy(a_hbm.at[di], buf, sem, add=True).wait()
            pltpu.async_copy(b_hbm.at[si], buf, sem, add=True).wait()

            def row(r, carry2):
                for cc in range(8):
                    sl = pl.ds(cc * 16, 16)
                    z = buf[r, sl]
                    buf[r, sl] = z / (1.0 + jnp.exp(-z))
                return carry2
            lax.fori_loop(0, ck, row, 0)
            pltpu.sync_copy(buf, aggr_sh.at[di], add=True)
            return carry
        lax.fori_loop(0, nchunk, chunk, 0)

        plsc.subcore_barrier()
        pltpu.sync_copy(aggr_sh.at[pl.ds(base, rows_per_tile)],
                        out_hbm.at[c, pl.ds(base, rows_per_tile)])

    return k(a_tab, b_tab, e4, dst3, src3)


def _sc_degrees(dst3, nn):
    nt, nchunk, ck = dst3.shape
    rows_per_tile = nn // NT
    zrows = 125

    @functools.partial(
        pl.kernel,
        out_type=jax.ShapeDtypeStruct((NCORE, nn, DW), F32),
        mesh=_sc_mesh(),
        compiler_params=pltpu.CompilerParams(use_tc_tiling_on_sc=False),
        scratch_types=[
            pltpu.VMEM((nchunk, ck), jnp.int32),
            pltpu.VMEM((ck, DW), F32),
            pltpu.VMEM((zrows, DW), F32),
            pltpu.VMEM_SHARED((nn, DW), F32),
        ],
    )
    def k(dst_hbm, out_hbm, dst_v, ones, zbuf, deg_sh):
        c = lax.axis_index("c")
        s = lax.axis_index("s")
        wid = c * NSUB + s

        def orow(r, carry):
            ones[r, pl.ds(0, 16)] = jnp.ones((16,), F32)
            return carry
        lax.fori_loop(0, ck, orow, 0)

        def zrow(r, carry):
            zbuf[r, pl.ds(0, 16)] = jnp.zeros((16,), F32)
            return carry
        lax.fori_loop(0, zrows, zrow, 0)

        base = s * rows_per_tile

        def zcp(p, carry):
            pltpu.sync_copy(zbuf, deg_sh.at[pl.ds(base + p * zrows, zrows)])
            return carry
        lax.fori_loop(0, rows_per_tile // zrows, zcp, 0)

        pltpu.sync_copy(dst_hbm.at[wid], dst_v)
        plsc.subcore_barrier()

        def chunk(j, carry):
            pltpu.sync_copy(ones, deg_sh.at[dst_v.at[j]], add=True)
            return carry
        lax.fori_loop(0, nchunk, chunk, 0)

        plsc.subcore_barrier()
        pltpu.sync_copy(deg_sh.at[pl.ds(base, rows_per_tile)],
                        out_hbm.at[c, pl.ds(base, rows_per_tile)])

    return k(dst3)


def kernel(x, edge_index, edge_attr, batch, params):
    n_nodes, node_f = x.shape
    n_edges = edge_index.shape[1]
    hidden = params["emb"][0].shape[1]
    nchunk = n_edges // (NT * CK)

    src3 = edge_index[0].reshape(NT, nchunk, CK)
    dst3 = edge_index[1].reshape(NT, nchunk, CK)

    h = _mm_bias(x, params["emb"][0], params["emb"][1], NB)
    deg = _sc_degrees(dst3, n_nodes)

    for lp in params["layers"]:
        wi = lp["eW1"][:hidden]
        wj = lp["eW1"][hidden:2 * hidden]
        we = lp["eW1"][2 * hidden:]
        w1a = lp["uW1"][:hidden]
        w1b = lp["uW1"][hidden:]

        a_tab, b_tab = _ab(h, wi, wj)
        e_rows = _mm_bias(edge_attr, we, lp["eb1"], EB)
        e4 = e_rows.reshape(NT, nchunk, CK, hidden)
        partials = _sc_edge_pass(a_tab, b_tab, e4, dst3, src3)
        h = _node_update(h, partials[0], partials[1], deg[0], deg[1],
                         lp["eW2"], lp["eb2"], w1a, w1b, lp["ub1"],
                         lp["uW2"], lp["ub2"])

    batch3 = batch.reshape(n_nodes // NB, 1, NB)
    n_graphs = 64
    return _pool_readout(batch3, h, n_graphs, params["R1"], params["rb1"],
                         params["R2"], params["rb2"], params["R3"],
                         params["rb3"])


# speedup vs baseline: 3.8657x; 1.3051x over previous
import functools

import jax
import jax.numpy as jnp
from jax import lax
from jax.experimental import pallas as pl
from jax.experimental.pallas import tpu as pltpu
from jax.experimental.pallas import tpu_sc as plsc

F32 = jnp.float32
NCORE = 2
NSUB = 16
NT = NCORE * NSUB
CK = 40
NB = 2000
EB = 4000
DW = 16


def _mm_bias_body(x_ref, w_ref, b_ref, o_ref):
    o_ref[...] = (
        jnp.dot(x_ref[...], w_ref[...], preferred_element_type=F32)
        + b_ref[...]
    )


def _mm_bias(x, w, b, rows_blk):
    n, kdim = x.shape
    fo = w.shape[1]
    return pl.pallas_call(
        _mm_bias_body,
        grid=(n // rows_blk,),
        in_specs=[
            pl.BlockSpec((rows_blk, kdim), lambda i: (i, 0)),
            pl.BlockSpec((kdim, fo), lambda i: (0, 0)),
            pl.BlockSpec((1, fo), lambda i: (0, 0)),
        ],
        out_specs=pl.BlockSpec((rows_blk, fo), lambda i: (i, 0)),
        out_shape=jax.ShapeDtypeStruct((n, fo), F32),
    )(x, w, b.reshape(1, fo))


def _ab_body(h_ref, wi_ref, wj_ref, a_ref, b_ref):
    h = h_ref[...]
    a_ref[...] = jnp.dot(h, wi_ref[...], preferred_element_type=F32)
    b_ref[...] = jnp.dot(h, wj_ref[...], preferred_element_type=F32)


def _ab(h, wi, wj):
    n, d = h.shape
    sds = jax.ShapeDtypeStruct((n, d), F32)
    return pl.pallas_call(
        _ab_body,
        grid=(n // NB,),
        in_specs=[
            pl.BlockSpec((NB, d), lambda i: (i, 0)),
            pl.BlockSpec((d, d), lambda i: (0, 0)),
            pl.BlockSpec((d, d), lambda i: (0, 0)),
        ],
        out_specs=[
            pl.BlockSpec((NB, d), lambda i: (i, 0)),
            pl.BlockSpec((NB, d), lambda i: (i, 0)),
        ],
        out_shape=[sds, sds],
    )(h, wi, wj)


def _node_body(h_ref, s0_ref, s1_ref, d0_ref, d1_ref, ew2_ref, eb2_ref,
               w1a_ref, w1b_ref, ub1_ref, w2_ref, ub2_ref, o_ref):
    s = s0_ref[...] + s1_ref[...]
    deg = (d0_ref[...] + d1_ref[...])[:, 0:1]
    aggr = (jnp.dot(s, ew2_ref[...], preferred_element_type=F32)
            + deg * eb2_ref[...])
    t = (jnp.dot(h_ref[...], w1a_ref[...], preferred_element_type=F32)
         + jnp.dot(aggr, w1b_ref[...], preferred_element_type=F32)
         + ub1_ref[...])
    t = t * jax.nn.sigmoid(t)
    y = jnp.dot(t, w2_ref[...], preferred_element_type=F32) + ub2_ref[...]
    o_ref[...] = y * jax.nn.sigmoid(y)


def _node_update(h, s0, s1, d0, d1, ew2, eb2, w1a, w1b, ub1, w2, ub2):
    n, d = h.shape
    return pl.pallas_call(
        _node_body,
        grid=(n // NB,),
        in_specs=[
            pl.BlockSpec((NB, d), lambda i: (i, 0)),
            pl.BlockSpec((NB, d), lambda i: (i, 0)),
            pl.BlockSpec((NB, d), lambda i: (i, 0)),
            pl.BlockSpec((NB, DW), lambda i: (i, 0)),
            pl.BlockSpec((NB, DW), lambda i: (i, 0)),
            pl.BlockSpec((d, d), lambda i: (0, 0)),
            pl.BlockSpec((1, d), lambda i: (0, 0)),
            pl.BlockSpec((d, d), lambda i: (0, 0)),
            pl.BlockSpec((d, d), lambda i: (0, 0)),
            pl.BlockSpec((1, d), lambda i: (0, 0)),
            pl.BlockSpec((d, d), lambda i: (0, 0)),
            pl.BlockSpec((1, d), lambda i: (0, 0)),
        ],
        out_specs=pl.BlockSpec((NB, d), lambda i: (i, 0)),
        out_shape=jax.ShapeDtypeStruct((n, d), F32),
    )(h, s0, s1, d0, d1, ew2, eb2.reshape(1, d), w1a, w1b,
      ub1.reshape(1, d), w2, ub2.reshape(1, d))


def _pool_body(batch_ref, h_ref, r1_ref, rb1_ref, r2_ref, rb2_ref, r3_ref,
               rb3_ref, o_ref, acc_ref, cnt_ref):
    i = pl.program_id(0)
    ng = acc_ref.shape[0]

    @pl.when(i == 0)
    def _():
        acc_ref[...] = jnp.zeros_like(acc_ref)
        cnt_ref[...] = jnp.zeros_like(cnt_ref)

    b = batch_ref[...].reshape(1, NB)
    mask = (lax.broadcasted_iota(jnp.int32, (ng, NB), 0) == b).astype(F32)
    acc_ref[...] += lax.dot_general(
        mask, h_ref[...], (((1,), (0,)), ((), ())),
        preferred_element_type=F32)
    cnt_ref[...] += jnp.sum(mask, axis=1, keepdims=True)

    @pl.when(i == pl.num_programs(0) - 1)
    def _():
        pooled = acc_ref[...] / jnp.maximum(cnt_ref[...], 1.0)
        t = jnp.dot(pooled, r1_ref[...], preferred_element_type=F32) + rb1_ref[...]
        t = t * jax.nn.sigmoid(t)
        t = jnp.dot(t, r2_ref[...], preferred_element_type=F32) + rb2_ref[...]
        t = t * jax.nn.sigmoid(t)
        o_ref[...] = jnp.dot(t, r3_ref[...], preferred_element_type=F32) + rb3_ref[...]


def _pool_readout(batch3, h, ng, r1, rb1, r2, rb2, r3, rb3):
    n, d = h.shape
    d2 = r2.shape[1]
    do = r3.shape[1]
    return pl.pallas_call(
        _pool_body,
        grid=(n // NB,),
        in_specs=[
            pl.BlockSpec((1, 1, NB), lambda i: (i, 0, 0)),
            pl.BlockSpec((NB, d), lambda i: (i, 0)),
            pl.BlockSpec((d, d), lambda i: (0, 0)),
            pl.BlockSpec((1, d), lambda i: (0, 0)),
            pl.BlockSpec((d, d2), lambda i: (0, 0)),
            pl.BlockSpec((1, d2), lambda i: (0, 0)),
            pl.BlockSpec((d2, do), lambda i: (0, 0)),
            pl.BlockSpec((1, do), lambda i: (0, 0)),
        ],
        out_specs=pl.BlockSpec((ng, do), lambda i: (0, 0)),
        out_shape=jax.ShapeDtypeStruct((ng, do), F32),
        scratch_shapes=[
            pltpu.VMEM((ng, d), F32),
            pltpu.VMEM((ng, 1), F32),
        ],
    )(batch3, h, r1, rb1.reshape(1, d), r2, rb2.reshape(1, d2), r3,
      rb3.reshape(1, do))


def _sc_mesh():
    return plsc.VectorSubcoreMesh(
        core_axis_name="c", subcore_axis_name="s",
        num_cores=NCORE, num_subcores=NSUB)


def _sc_edge_pass(a_tab, b_tab, e4, dst3, src3):
    nt, nchunk, ck = dst3.shape
    nn, d = a_tab.shape
    rows_per_tile = nn // NSUB
    zrows = 25
    gsz = 50
    ngrp = nchunk // gsz

    @functools.partial(
        pl.kernel,
        out_type=jax.ShapeDtypeStruct((NCORE, nn, d), F32),
        mesh=_sc_mesh(),
        compiler_params=pltpu.CompilerParams(use_tc_tiling_on_sc=False),
        scratch_types=[
            pltpu.VMEM((gsz, ck), jnp.int32),
            pltpu.VMEM((gsz, ck), jnp.int32),
            pltpu.VMEM((ck, 128), F32),
            pltpu.VMEM((ck, 128), F32),
            pltpu.VMEM((ck, 128), F32),
            pltpu.VMEM((ck, 128), F32),
            pltpu.VMEM((ck, 128), F32),
            pltpu.VMEM((ck, 128), F32),
            pltpu.VMEM((zrows, 128), F32),
            pltpu.VMEM_SHARED((nn, 128), F32),
            pltpu.SemaphoreType.DMA,
        ],
    )
    def k(a_hbm, b_hbm, e_hbm, dst_hbm, src_hbm, out_hbm,
          dst_v, src_v, e0, a0, b0, e1, a1, b1, zbuf, aggr_sh, sem):
        c = lax.axis_index("c")
        s = lax.axis_index("s")
        wid = c * NSUB + s

        def zrow(r, carry):
            for cc in range(8):
                zbuf[r, pl.ds(cc * 16, 16)] = jnp.zeros((16,), F32)
            return carry
        lax.fori_loop(0, zrows, zrow, 0)

        base = s * rows_per_tile

        def zcp(p, carry):
            pltpu.sync_copy(zbuf, aggr_sh.at[pl.ds(base + p * zrows, zrows)])
            return carry
        lax.fori_loop(0, rows_per_tile // zrows, zcp, 0)
        plsc.subcore_barrier()

        def silu_scatter(eb, ab, bb, idx):
            def row(r, carry2):
                for cc in range(8):
                    sl = pl.ds(cc * 16, 16)
                    z = eb[r, sl] + ab[r, sl] + bb[r, sl]
                    eb[r, sl] = z / (1.0 + jnp.exp(-z))
                return carry2
            lax.fori_loop(0, ck, row, 0)
            pltpu.sync_copy(eb, aggr_sh.at[idx], add=True)

        def group(g, carry):
            pltpu.sync_copy(dst_hbm.at[wid, pl.ds(g * gsz, gsz)], dst_v)
            pltpu.sync_copy(src_hbm.at[wid, pl.ds(g * gsz, gsz)], src_v)

            def pair(t, carry2):
                j0 = 2 * t
                j1 = j0 + 1
                gc0 = g * gsz + j0
                di0 = dst_v.at[j0]
                di1 = dst_v.at[j1]
                ce0 = pltpu.async_copy(e_hbm.at[wid, gc0], e0, sem)
                ca0 = pltpu.async_copy(a_hbm.at[di0], a0, sem)
                cb0 = pltpu.async_copy(b_hbm.at[src_v.at[j0]], b0, sem)
                ce1 = pltpu.async_copy(e_hbm.at[wid, gc0 + 1], e1, sem)
                ca1 = pltpu.async_copy(a_hbm.at[di1], a1, sem)
                cb1 = pltpu.async_copy(b_hbm.at[src_v.at[j1]], b1, sem)
                ce0.wait()
                ca0.wait()
                cb0.wait()
                silu_scatter(e0, a0, b0, di0)
                ce1.wait()
                ca1.wait()
                cb1.wait()
                silu_scatter(e1, a1, b1, di1)
                return carry2
            lax.fori_loop(0, gsz // 2, pair, 0)
            return carry
        lax.fori_loop(0, ngrp, group, 0)

        plsc.subcore_barrier()
        pltpu.sync_copy(aggr_sh.at[pl.ds(base, rows_per_tile)],
                        out_hbm.at[c, pl.ds(base, rows_per_tile)])

    return k(a_tab, b_tab, e4, dst3, src3)


def _sc_degrees(dst3, nn):
    nt, nchunk, ck = dst3.shape
    rows_per_tile = nn // NSUB
    zrows = 125

    @functools.partial(
        pl.kernel,
        out_type=jax.ShapeDtypeStruct((NCORE, nn, DW), F32),
        mesh=_sc_mesh(),
        compiler_params=pltpu.CompilerParams(use_tc_tiling_on_sc=False),
        scratch_types=[
            pltpu.VMEM((nchunk, ck), jnp.int32),
            pltpu.VMEM((ck, DW), F32),
            pltpu.VMEM((zrows, DW), F32),
            pltpu.VMEM_SHARED((nn, DW), F32),
        ],
    )
    def k(dst_hbm, out_hbm, dst_v, ones, zbuf, deg_sh):
        c = lax.axis_index("c")
        s = lax.axis_index("s")
        wid = c * NSUB + s

        def orow(r, carry):
            ones[r, pl.ds(0, 16)] = jnp.ones((16,), F32)
            return carry
        lax.fori_loop(0, ck, orow, 0)

        def zrow(r, carry):
            zbuf[r, pl.ds(0, 16)] = jnp.zeros((16,), F32)
            return carry
        lax.fori_loop(0, zrows, zrow, 0)

        base = s * rows_per_tile

        def zcp(p, carry):
            pltpu.sync_copy(zbuf, deg_sh.at[pl.ds(base + p * zrows, zrows)])
            return carry
        lax.fori_loop(0, rows_per_tile // zrows, zcp, 0)

        pltpu.sync_copy(dst_hbm.at[wid], dst_v)
        plsc.subcore_barrier()

        def chunk(j, carry):
            pltpu.sync_copy(ones, deg_sh.at[dst_v.at[j]], add=True)
            return carry
        lax.fori_loop(0, nchunk, chunk, 0)

        plsc.subcore_barrier()
        pltpu.sync_copy(deg_sh.at[pl.ds(base, rows_per_tile)],
                        out_hbm.at[c, pl.ds(base, rows_per_tile)])

    return k(dst3)


def kernel(x, edge_index, edge_attr, batch, params):
    n_nodes, node_f = x.shape
    n_edges = edge_index.shape[1]
    hidden = params["emb"][0].shape[1]
    nchunk = n_edges // (NT * CK)

    src3 = edge_index[0].reshape(NT, nchunk, CK)
    dst3 = edge_index[1].reshape(NT, nchunk, CK)

    h = _mm_bias(x, params["emb"][0], params["emb"][1], NB)
    deg = _sc_degrees(dst3, n_nodes)

    for lp in params["layers"]:
        wi = lp["eW1"][:hidden]
        wj = lp["eW1"][hidden:2 * hidden]
        we = lp["eW1"][2 * hidden:]
        w1a = lp["uW1"][:hidden]
        w1b = lp["uW1"][hidden:]

        a_tab, b_tab = _ab(h, wi, wj)
        e_rows = _mm_bias(edge_attr, we, lp["eb1"], EB)
        e4 = e_rows.reshape(NT, nchunk, CK, hidden)
        partials = _sc_edge_pass(a_tab, b_tab, e4, dst3, src3)
        h = _node_update(h, partials[0], partials[1], deg[0], deg[1],
                         lp["eW2"], lp["eb2"], w1a, w1b, lp["ub1"],
                         lp["uW2"], lp["ub2"])

    batch3 = batch.reshape(n_nodes // NB, 1, NB)
    n_graphs = 64
    return _pool_readout(batch3, h, n_graphs, params["R1"], params["rb1"],
                         params["R2"], params["rb2"], params["R3"],
                         params["rb3"])
